# baseline (speedup 1.0000x reference)
import functools

import jax
import jax.numpy as jnp
from jax import lax
from jax.experimental import pallas as pl
from jax.experimental.pallas import tpu as pltpu
from jax.experimental.pallas import tpu_sc as plsc

B = 1024
K = 16
D = 32
NUM_REL = 32
NUM_ENT = 100000
NUM_USR = 10000

NC = 2
NS = 16
NW = NC * NS
BPW = B // NW
NG = BPW // 16
TPW = BPW // 8


def _sc_fused_kernel():
  mesh = plsc.VectorSubcoreMesh(
      core_axis_name="c", subcore_axis_name="s",
      num_cores=NC, num_subcores=NS)

  @functools.partial(
      pl.kernel,
      mesh=mesh,
      compiler_params=pltpu.CompilerParams(use_tc_tiling_on_sc=True,
                                           needs_layout_passes=False),
      out_type=jax.ShapeDtypeStruct((B,), jnp.float32),
      scratch_types=[
          pltpu.VMEM((BPW,), jnp.int32),
          pltpu.VMEM((BPW,), jnp.int32),
          pltpu.VMEM((TPW, 8, K), jnp.int32),
          pltpu.VMEM((TPW, 8, K), jnp.int32),
          pltpu.VMEM((TPW, 8, D), jnp.float32),
          pltpu.VMEM((TPW, 8, D), jnp.float32),
          pltpu.VMEM((BPW * K // 8, 8, D), jnp.float32),
          pltpu.VMEM((D, 16), jnp.float32),
          pltpu.VMEM((D, 16), jnp.float32),
          pltpu.VMEM((BPW,), jnp.float32),
          pltpu.SemaphoreType.DMA,
          pltpu.SemaphoreType.DMA,
          pltpu.SemaphoreType.DMA,
          pltpu.SemaphoreType.DMA,
      ],
  )
  def sc_fused(u_h, v_h, ae_h, ar_h, usr_h, ent_h, relT_h, wT_h, b_h,
               out_h, uix, vix, aeb, arb, ueb, vsb, neb, uet, xt, res,
               sem_r, sem_a, sem_n0, sem_n1):
    wid = lax.axis_index("s") * NC + lax.axis_index("c")
    base = wid * BPW
    pltpu.sync_copy(u_h.at[pl.ds(base, BPW)], uix)
    pltpu.sync_copy(v_h.at[pl.ds(base, BPW)], vix)
    def fetch_rows(c, carry):
      uvec = uix[pl.ds(c * 16, 16)]
      vvec = vix[pl.ds(c * 16, 16)]
      for l in range(16):
        j = c * 16 + l
        tj, sj = j >> 3, j & 7
        vv = vvec[l]
        uu = uvec[l]
        vt, vs2 = vv >> 3, vv & 7
        pltpu.async_copy(ae_h.at[vt, vs2], aeb.at[tj, sj], sem_a)
        pltpu.async_copy(ar_h.at[vt, vs2], arb.at[tj, sj], sem_a)
        pltpu.async_copy(usr_h.at[uu >> 3, uu & 7], ueb.at[tj, sj], sem_r)
        pltpu.async_copy(ent_h.at[vt, vs2], vsb.at[tj, sj], sem_r)
      return carry
    lax.fori_loop(0, NG, fetch_rows, 0)
    def drain_ae(j, carry):
      pltpu.make_async_copy(ae_h.at[0, 0], aeb.at[j >> 3, j & 7],
                            sem_a).wait()
      return carry
    lax.fori_loop(0, BPW, drain_ae, 0)
    for c, sem_n in ((0, sem_n0), (1, sem_n1)):
      for l in range(16):
        j = c * 16 + l
        row16 = aeb[j >> 3, j & 7]
        for k in range(K):
          e = row16[k]
          r = j * K + k
          pltpu.async_copy(ent_h.at[e >> 3, e & 7],
                           neb.at[r >> 3, r & 7], sem_n)
    def drain_rest(j, carry):
      tj, sj = j >> 3, j & 7
      pltpu.make_async_copy(ar_h.at[0, 0], arb.at[tj, sj], sem_a).wait()
      pltpu.make_async_copy(usr_h.at[0, 0], ueb.at[tj, sj], sem_r).wait()
      pltpu.make_async_copy(ent_h.at[0, 0], vsb.at[tj, sj], sem_r).wait()
      return carry
    lax.fori_loop(0, BPW, drain_rest, 0)
    pl.run_scoped(
        lambda relT_v, wT_v, b_v: _sc_math(
            aeb, arb, ueb, vsb, neb, uet, xt, res,
            relT_v, wT_v, b_v, relT_h, wT_h, b_h, ent_h, out_h, base,
            (sem_n0, sem_n1)),
        pltpu.VMEM((D, NUM_REL), jnp.float32),
        pltpu.VMEM((D, D), jnp.float32),
        pltpu.VMEM((D,), jnp.float32),
    )

  def _sc_math(aeb, arb, ueb, vsb, neb, uet, xt, res,
               relT_v, wT_v, b_v, relT_h, wT_h, b_h, ent_h, out_h, base,
               sem_ns):
    pltpu.sync_copy(relT_h, relT_v)
    pltpu.sync_copy(wT_h, wT_v)
    pltpu.sync_copy(b_h, b_v)
    lanes = lax.iota(jnp.int32, 16)
    for g in range(NG):
      def drain_ne(j, carry):
        for k in range(K):
          r = j * K + k
          pltpu.make_async_copy(ent_h.at[0, 0], neb.at[r >> 3, r & 7],
                                sem_ns[g]).wait()
        return carry
      lax.fori_loop(g * 16, (g + 1) * 16, drain_ne, 0)
      bvec = lanes + g * 16
      bt, bs = bvec >> 3, bvec & 7
      relids = [plsc.load_gather(arb, [bt, bs, lanes * 0 + k])
                for k in range(K)]
      def logits_step(d, s):
        dvec = lanes * 0 + d
        ued = plsc.load_gather(ueb, [bt, bs, dvec])
        uet[d] = ued
        return tuple(
            s[k] + ued * plsc.load_gather(relT_v, [dvec, relids[k]])
            for k in range(K))
      s = lax.fori_loop(
          0, D, logits_step,
          tuple(jnp.zeros((16,), jnp.float32) for _ in range(K)))
      m = s[0]
      for k in range(1, K):
        m = jnp.maximum(m, s[k])
      es = [jnp.exp(s[k] - m) for k in range(K)]
      tot = es[0]
      for k in range(1, K):
        tot = tot + es[k]
      inv = 1.0 / tot
      p = [es[k] * inv for k in range(K)]
      nrows = [bvec * K + k for k in range(K)]
      nts = [(nrows[k] >> 3, nrows[k] & 7) for k in range(K)]
      def wsum_step(d, carry):
        dvec = lanes * 0 + d
        acc = plsc.load_gather(vsb, [bt, bs, dvec])
        for k in range(K):
          ned = plsc.load_gather(neb, [nts[k][0], nts[k][1], dvec])
          acc = acc + p[k] * ned
        xt[d] = acc
        return carry
      lax.fori_loop(0, D, wsum_step, 0)
      def lin_step(do, y):
        dovec = lanes * 0 + do
        accw = plsc.load_gather(b_v, [dovec])
        for j in range(D):
          wv = plsc.load_gather(wT_v, [lanes * 0 + j, dovec])
          accw = accw + xt[j] * wv
        vu = jnp.maximum(accw, 0.0)
        return y + plsc.load_gather(uet, [dovec, lanes]) * vu
      y = lax.fori_loop(0, D, lin_step, jnp.zeros((16,), jnp.float32))
      res[pl.ds(g * 16, 16)] = 1.0 / (1.0 + jnp.exp(-y))
    pltpu.sync_copy(res, out_h.at[pl.ds(base, BPW)])

  return sc_fused


def kernel(u, v, adj_ent, adj_rel, usr_emb, ent_emb, rel_emb, W, b):
  ae3 = adj_ent.astype(jnp.int32).reshape(NUM_ENT // 8, 8, K)
  ar3 = adj_rel.astype(jnp.int32).reshape(NUM_ENT // 8, 8, K)
  usr3 = usr_emb.reshape(NUM_USR // 8, 8, D)
  ent3 = ent_emb.reshape(NUM_ENT // 8, 8, D)
  return _sc_fused_kernel()(
      u.astype(jnp.int32), v.astype(jnp.int32), ae3, ar3, usr3, ent3,
      rel_emb.T, W.T, b)


# speedup vs baseline: 1.0506x; 1.0506x over previous
import functools

import jax
import jax.numpy as jnp
from jax import lax
from jax.experimental import pallas as pl
from jax.experimental.pallas import tpu as pltpu
from jax.experimental.pallas import tpu_sc as plsc

B = 1024
K = 16
D = 32
NUM_REL = 32
NUM_ENT = 100000
NUM_USR = 10000

NC = 2
NS = 16
NW = NC * NS
BPW = B // NW
NG = BPW // 16
TPW = BPW // 8


def _sc_fused_kernel():
  mesh = plsc.VectorSubcoreMesh(
      core_axis_name="c", subcore_axis_name="s",
      num_cores=NC, num_subcores=NS)

  @functools.partial(
      pl.kernel,
      mesh=mesh,
      compiler_params=pltpu.CompilerParams(use_tc_tiling_on_sc=True,
                                           needs_layout_passes=False),
      out_type=jax.ShapeDtypeStruct((B,), jnp.float32),
      scratch_types=[
          pltpu.VMEM((BPW,), jnp.int32),
          pltpu.VMEM((BPW,), jnp.int32),
          pltpu.VMEM((TPW, 8, K), jnp.int32),
          pltpu.VMEM((TPW, 8, K), jnp.int32),
          pltpu.VMEM((TPW, 8, D), jnp.float32),
          pltpu.VMEM((TPW, 8, D), jnp.float32),
          pltpu.VMEM((BPW * K // 8, 8, D), jnp.float32),
          pltpu.VMEM((D, 16), jnp.float32),
          pltpu.VMEM((D, 16), jnp.float32),
          pltpu.VMEM((BPW,), jnp.float32),
          pltpu.SemaphoreType.DMA,
          pltpu.SemaphoreType.DMA,
          pltpu.SemaphoreType.DMA,
          pltpu.SemaphoreType.DMA,
      ],
  )
  def sc_fused(u_h, v_h, ae_h, ar_h, usr_h, ent_h, relT_h, wT_h, b_h,
               out_h, uix, vix, aeb, arb, ueb, vsb, neb, uet, xt, res,
               sem_r, sem_a, sem_n0, sem_n1):
    wid = lax.axis_index("s") * NC + lax.axis_index("c")
    base = wid * BPW
    pltpu.sync_copy(u_h.at[pl.ds(base, BPW)], uix)
    pltpu.sync_copy(v_h.at[pl.ds(base, BPW)], vix)
    def fetch_rows(c, carry):
      uvec = uix[pl.ds(c * 16, 16)]
      vvec = vix[pl.ds(c * 16, 16)]
      for l in range(16):
        j = c * 16 + l
        tj, sj = j >> 3, j & 7
        vv = vvec[l]
        uu = uvec[l]
        vt, vs2 = vv >> 3, vv & 7
        pltpu.async_copy(ae_h.at[vt, vs2], aeb.at[tj, sj], sem_a)
        pltpu.async_copy(ar_h.at[vt, vs2], arb.at[tj, sj], sem_a)
        pltpu.async_copy(usr_h.at[uu >> 3, uu & 7], ueb.at[tj, sj], sem_r)
        pltpu.async_copy(ent_h.at[vt, vs2], vsb.at[tj, sj], sem_r)
      return carry
    lax.fori_loop(0, NG, fetch_rows, 0)
    def drain_ae(j, carry):
      pltpu.make_async_copy(ae_h.at[0, 0], aeb.at[j >> 3, j & 7],
                            sem_a).wait()
      return carry
    lax.fori_loop(0, BPW, drain_ae, 0)
    for sem_n, lo in ((sem_n0, 0), (sem_n1, 16)):
      def fetch_ne(j, carry, sem_n=sem_n):
        row16 = aeb[j >> 3, j & 7]
        for k in range(K):
          e = row16[k]
          r = j * K + k
          pltpu.async_copy(ent_h.at[e >> 3, e & 7],
                           neb.at[r >> 3, r & 7], sem_n)
        return carry
      lax.fori_loop(lo, lo + 16, fetch_ne, 0)
    def drain_rest(j, carry):
      tj, sj = j >> 3, j & 7
      pltpu.make_async_copy(ar_h.at[0, 0], arb.at[tj, sj], sem_a).wait()
      pltpu.make_async_copy(usr_h.at[0, 0], ueb.at[tj, sj], sem_r).wait()
      pltpu.make_async_copy(ent_h.at[0, 0], vsb.at[tj, sj], sem_r).wait()
      return carry
    lax.fori_loop(0, BPW, drain_rest, 0)
    pl.run_scoped(
        lambda relT_v, wT_v, b_v: _sc_math(
            aeb, arb, ueb, vsb, neb, uet, xt, res,
            relT_v, wT_v, b_v, relT_h, wT_h, b_h, ent_h, out_h, base,
            (sem_n0, sem_n1)),
        pltpu.VMEM((D, NUM_REL), jnp.float32),
        pltpu.VMEM((D, D), jnp.float32),
        pltpu.VMEM((D,), jnp.float32),
    )

  def _sc_math(aeb, arb, ueb, vsb, neb, uet, xt, res,
               relT_v, wT_v, b_v, relT_h, wT_h, b_h, ent_h, out_h, base,
               sem_ns):
    pltpu.sync_copy(relT_h, relT_v)
    pltpu.sync_copy(wT_h, wT_v)
    pltpu.sync_copy(b_h, b_v)
    lanes = lax.iota(jnp.int32, 16)
    for g in range(NG):
      def drain_ne(j, carry):
        for k in range(K):
          r = j * K + k
          pltpu.make_async_copy(ent_h.at[0, 0], neb.at[r >> 3, r & 7],
                                sem_ns[g]).wait()
        return carry
      lax.fori_loop(g * 16, (g + 1) * 16, drain_ne, 0)
      bvec = lanes + g * 16
      bt, bs = bvec >> 3, bvec & 7
      relids = [plsc.load_gather(arb, [bt, bs, lanes * 0 + k])
                for k in range(K)]
      def logits_step(d, s):
        dvec = lanes * 0 + d
        ued = plsc.load_gather(ueb, [bt, bs, dvec])
        uet[d] = ued
        return tuple(
            s[k] + ued * plsc.load_gather(relT_v, [dvec, relids[k]])
            for k in range(K))
      s = lax.fori_loop(
          0, D, logits_step,
          tuple(jnp.zeros((16,), jnp.float32) for _ in range(K)))
      m = s[0]
      for k in range(1, K):
        m = jnp.maximum(m, s[k])
      es = [jnp.exp(s[k] - m) for k in range(K)]
      tot = es[0]
      for k in range(1, K):
        tot = tot + es[k]
      inv = 1.0 / tot
      p = [es[k] * inv for k in range(K)]
      nrows = [bvec * K + k for k in range(K)]
      nts = [(nrows[k] >> 3, nrows[k] & 7) for k in range(K)]
      def wsum_step(d, carry):
        dvec = lanes * 0 + d
        acc = plsc.load_gather(vsb, [bt, bs, dvec])
        for k in range(K):
          ned = plsc.load_gather(neb, [nts[k][0], nts[k][1], dvec])
          acc = acc + p[k] * ned
        xt[d] = acc
        return carry
      lax.fori_loop(0, D, wsum_step, 0)
      def lin_step(do, y):
        dovec = lanes * 0 + do
        accw = plsc.load_gather(b_v, [dovec])
        for j in range(D):
          wv = plsc.load_gather(wT_v, [lanes * 0 + j, dovec])
          accw = accw + xt[j] * wv
        vu = jnp.maximum(accw, 0.0)
        return y + plsc.load_gather(uet, [dovec, lanes]) * vu
      y = lax.fori_loop(0, D, lin_step, jnp.zeros((16,), jnp.float32))
      res[pl.ds(g * 16, 16)] = 1.0 / (1.0 + jnp.exp(-y))
    pltpu.sync_copy(res, out_h.at[pl.ds(base, BPW)])

  return sc_fused


def kernel(u, v, adj_ent, adj_rel, usr_emb, ent_emb, rel_emb, W, b):
  ae3 = adj_ent.astype(jnp.int32).reshape(NUM_ENT // 8, 8, K)
  ar3 = adj_rel.astype(jnp.int32).reshape(NUM_ENT // 8, 8, K)
  usr3 = usr_emb.reshape(NUM_USR // 8, 8, D)
  ent3 = ent_emb.reshape(NUM_ENT // 8, 8, D)
  return _sc_fused_kernel()(
      u.astype(jnp.int32), v.astype(jnp.int32), ae3, ar3, usr3, ent3,
      rel_emb.T, W.T, b)


# speedup vs baseline: 1.0513x; 1.0007x over previous
import functools

import jax
import jax.numpy as jnp
from jax import lax
from jax.experimental import pallas as pl
from jax.experimental.pallas import tpu as pltpu
from jax.experimental.pallas import tpu_sc as plsc

B = 1024
K = 16
D = 32
NUM_REL = 32
NUM_ENT = 100000
NUM_USR = 10000

NC = 2
NS = 16
NW = NC * NS
BPW = B // NW
NG = BPW // 16
TPW = BPW // 8


def _sc_fused_kernel():
  mesh = plsc.VectorSubcoreMesh(
      core_axis_name="c", subcore_axis_name="s",
      num_cores=NC, num_subcores=NS)

  @functools.partial(
      pl.kernel,
      mesh=mesh,
      compiler_params=pltpu.CompilerParams(use_tc_tiling_on_sc=True,
                                           needs_layout_passes=False),
      out_type=jax.ShapeDtypeStruct((B,), jnp.float32),
      scratch_types=[
          pltpu.VMEM((BPW,), jnp.int32),
          pltpu.VMEM((BPW,), jnp.int32),
          pltpu.VMEM((TPW, 8, K), jnp.int32),
          pltpu.VMEM((TPW, 8, K), jnp.int32),
          pltpu.VMEM((TPW, 8, D), jnp.float32),
          pltpu.VMEM((TPW, 8, D), jnp.float32),
          pltpu.VMEM((BPW * K // 8, 8, D), jnp.float32),
          pltpu.VMEM((D, 16), jnp.float32),
          pltpu.VMEM((D, 16), jnp.float32),
          pltpu.VMEM((BPW,), jnp.float32),
          pltpu.SemaphoreType.DMA,
          pltpu.SemaphoreType.DMA,
          pltpu.SemaphoreType.DMA,
          pltpu.SemaphoreType.DMA,
      ],
  )
  def sc_fused(u_h, v_h, ae_h, ar_h, usr_h, ent_h, relT_h, wT_h, b_h,
               out_h, uix, vix, aeb, arb, ueb, vsb, neb, uet, xt, res,
               sem_r, sem_a, sem_n0, sem_n1):
    wid = lax.axis_index("s") * NC + lax.axis_index("c")
    base = wid * BPW
    pltpu.sync_copy(u_h.at[pl.ds(base, BPW)], uix)
    pltpu.sync_copy(v_h.at[pl.ds(base, BPW)], vix)
    def fetch_rows(c, carry):
      uvec = uix[pl.ds(c * 16, 16)]
      vvec = vix[pl.ds(c * 16, 16)]
      for l in range(16):
        j = c * 16 + l
        tj, sj = j >> 3, j & 7
        vv = vvec[l]
        uu = uvec[l]
        vt, vs2 = vv >> 3, vv & 7
        pltpu.async_copy(ae_h.at[vt, vs2], aeb.at[tj, sj], sem_a)
        pltpu.async_copy(ar_h.at[vt, vs2], arb.at[tj, sj], sem_a)
        pltpu.async_copy(usr_h.at[uu >> 3, uu & 7], ueb.at[tj, sj], sem_r)
        pltpu.async_copy(ent_h.at[vt, vs2], vsb.at[tj, sj], sem_r)
      return carry
    lax.fori_loop(0, NG, fetch_rows, 0)
    def drain_ae(j, carry):
      pltpu.make_async_copy(ae_h.at[0, 0], aeb.at[j >> 3, j & 7],
                            sem_a).wait()
      return carry
    lax.fori_loop(0, BPW, drain_ae, 0)
    for sem_n, lo in ((sem_n0, 0), (sem_n1, 16)):
      def fetch_ne(j, carry, sem_n=sem_n):
        row16 = aeb[j >> 3, j & 7]
        for k in range(K):
          e = row16[k]
          r = j * K + k
          pltpu.async_copy(ent_h.at[e >> 3, e & 7],
                           neb.at[r >> 3, r & 7], sem_n)
        return carry
      lax.fori_loop(lo, lo + 16, fetch_ne, 0)
    def drain_rest(j, carry):
      tj, sj = j >> 3, j & 7
      pltpu.make_async_copy(ar_h.at[0, 0], arb.at[tj, sj], sem_a).wait()
      pltpu.make_async_copy(usr_h.at[0, 0], ueb.at[tj, sj], sem_r).wait()
      pltpu.make_async_copy(ent_h.at[0, 0], vsb.at[tj, sj], sem_r).wait()
      return carry
    lax.fori_loop(0, BPW, drain_rest, 0)
    pl.run_scoped(
        lambda relT_v, wT_v, b_v: _sc_math(
            aeb, arb, ueb, vsb, neb, uet, xt, res,
            relT_v, wT_v, b_v, relT_h, wT_h, b_h, ent_h, out_h, base,
            (sem_n0, sem_n1)),
        pltpu.VMEM((D, NUM_REL), jnp.float32),
        pltpu.VMEM((D, D), jnp.float32),
        pltpu.VMEM((D,), jnp.float32),
    )

  def _sc_math(aeb, arb, ueb, vsb, neb, uet, xt, res,
               relT_v, wT_v, b_v, relT_h, wT_h, b_h, ent_h, out_h, base,
               sem_ns):
    pltpu.sync_copy(relT_h, relT_v)
    pltpu.sync_copy(wT_h, wT_v)
    pltpu.sync_copy(b_h, b_v)
    lanes = lax.iota(jnp.int32, 16)
    for g in range(NG):
      def drain_ne(j, carry):
        for k in range(K):
          r = j * K + k
          pltpu.make_async_copy(ent_h.at[0, 0], neb.at[r >> 3, r & 7],
                                sem_ns[g]).wait()
        return carry
      lax.fori_loop(g * 16, (g + 1) * 16, drain_ne, 0)
      bvec = lanes + g * 16
      bt, bs = bvec >> 3, bvec & 7
      relids = [plsc.load_gather(arb, [bt, bs, lanes * 0 + k])
                for k in range(K)]
      def logits_step(d, s):
        dvec = lanes * 0 + d
        ued = plsc.load_gather(ueb, [bt, bs, dvec])
        uet[d] = ued
        return tuple(
            s[k] + ued * plsc.load_gather(relT_v, [dvec, relids[k]])
            for k in range(K))
      s = lax.fori_loop(
          0, D, logits_step,
          tuple(jnp.zeros((16,), jnp.float32) for _ in range(K)))
      m = s[0]
      for k in range(1, K):
        m = jnp.maximum(m, s[k])
      es = [jnp.exp(s[k] - m) for k in range(K)]
      tot = es[0]
      for k in range(1, K):
        tot = tot + es[k]
      inv = 1.0 / tot
      p = [es[k] * inv for k in range(K)]
      nrows = [bvec * K + k for k in range(K)]
      nts = [(nrows[k] >> 3, nrows[k] & 7) for k in range(K)]
      def wsum_step(d, carry):
        dvec = lanes * 0 + d
        acc = plsc.load_gather(vsb, [bt, bs, dvec])
        for k in range(K):
          ned = plsc.load_gather(neb, [nts[k][0], nts[k][1], dvec])
          acc = acc + p[k] * ned
        xt[d] = acc
        return carry
      lax.fori_loop(0, D, wsum_step, 0)
      def lin_step(do, y):
        dovec = lanes * 0 + do
        accw = plsc.load_gather(b_v, [dovec])
        for j in range(D):
          wv = plsc.load_gather(wT_v, [lanes * 0 + j, dovec])
          accw = accw + xt[j] * wv
        vu = jnp.maximum(accw, 0.0)
        return y + uet[do] * vu
      y = lax.fori_loop(0, D, lin_step, jnp.zeros((16,), jnp.float32))
      res[pl.ds(g * 16, 16)] = 1.0 / (1.0 + jnp.exp(-y))
    pltpu.sync_copy(res, out_h.at[pl.ds(base, BPW)])

  return sc_fused


def kernel(u, v, adj_ent, adj_rel, usr_emb, ent_emb, rel_emb, W, b):
  ae3 = adj_ent.astype(jnp.int32).reshape(NUM_ENT // 8, 8, K)
  ar3 = adj_rel.astype(jnp.int32).reshape(NUM_ENT // 8, 8, K)
  usr3 = usr_emb.reshape(NUM_USR // 8, 8, D)
  ent3 = ent_emb.reshape(NUM_ENT // 8, 8, D)
  return _sc_fused_kernel()(
      u.astype(jnp.int32), v.astype(jnp.int32), ae3, ar3, usr3, ent3,
      rel_emb.T, W.T, b)


# speedup vs baseline: 1.3276x; 1.2629x over previous
import functools

import jax
import jax.numpy as jnp
from jax import lax
from jax.experimental import pallas as pl
from jax.experimental.pallas import tpu as pltpu
from jax.experimental.pallas import tpu_sc as plsc

B = 1024
K = 16
D = 32
NUM_REL = 32
NUM_ENT = 100000
NUM_USR = 10000

NC = 2
NS = 16
NW = NC * NS
BPW = B // NW
NG = BPW // 16
TPW = BPW // 8


def _sc_fused_kernel():
  mesh = plsc.VectorSubcoreMesh(
      core_axis_name="c", subcore_axis_name="s",
      num_cores=NC, num_subcores=NS)

  @functools.partial(
      pl.kernel,
      mesh=mesh,
      compiler_params=pltpu.CompilerParams(use_tc_tiling_on_sc=True,
                                           needs_layout_passes=False),
      out_type=jax.ShapeDtypeStruct((B,), jnp.float32),
      scratch_types=[
          pltpu.VMEM((BPW,), jnp.int32),
          pltpu.VMEM((BPW,), jnp.int32),
          pltpu.VMEM((TPW, 8, K), jnp.int32),
          pltpu.VMEM((TPW, 8, K), jnp.int32),
          pltpu.VMEM((TPW, 8, D), jnp.float32),
          pltpu.VMEM((TPW, 8, D), jnp.float32),
          pltpu.VMEM((D, 16), jnp.float32),
          pltpu.VMEM((D, 16), jnp.float32),
          pltpu.VMEM((BPW,), jnp.float32),
          pltpu.SemaphoreType.DMA,
          pltpu.SemaphoreType.DMA,
          pltpu.SemaphoreType.DMA,
          pltpu.SemaphoreType.DMA,
      ],
  )
  def sc_fused(u_h, v_h, aeT_h, arT_h, usr_h, ent_h, relT_h, wT_h, b_h,
               out_h, uix, vix, aeb, arb, ueb, vsb, uet, xt, res,
               sem_r, sem_a, sem_n0, sem_n1):
    wid = lax.axis_index("s") * NC + lax.axis_index("c")
    base = wid * BPW
    pltpu.sync_copy(u_h.at[pl.ds(base, BPW)], uix)
    pltpu.sync_copy(v_h.at[pl.ds(base, BPW)], vix)
    def fetch_rows(c, carry):
      uvec = uix[pl.ds(c * 16, 16)]
      vvec = vix[pl.ds(c * 16, 16)]
      for l in range(16):
        j = c * 16 + l
        tj, sj = j >> 3, j & 7
        vv = vvec[l]
        uu = uvec[l]
        pltpu.async_copy(usr_h.at[uu >> 3, uu & 7], ueb.at[tj, sj], sem_r)
        pltpu.async_copy(ent_h.at[vv >> 3, vv & 7], vsb.at[tj, sj], sem_r)
      return carry
    lax.fori_loop(0, NG, fetch_rows, 0)
    lanes = lax.iota(jnp.int32, 16)
    def adj_phase(adjN):
      for src, dst in ((aeT_h, aeb), (arT_h, arb)):
        def fetch_adj(c, carry, src=src):
          vvec = vix[pl.ds(c * 16, 16)]
          for l in range(16):
            j = c * 16 + l
            off = pl.multiple_of((vvec[l] >> 7) * 128, 128)
            for g2 in range(2):
              pltpu.async_copy(src.at[g2, :, pl.ds(off, 128)],
                               adjN.at[2 * j + g2], sem_a)
          return carry
        lax.fori_loop(0, NG, fetch_adj, 0)
        def drain_adj(j, carry):
          for g2 in range(2):
            pltpu.make_async_copy(src.at[g2, :, pl.ds(0, 128)],
                                  adjN.at[2 * j + g2], sem_a).wait()
          return carry
        lax.fori_loop(0, BPW, drain_adj, 0)
        def extract_adj(c, carry, dst=dst):
          vvec = vix[pl.ds(c * 16, 16)]
          for l in range(16):
            j = c * 16 + l
            lvec = lanes * 0 + (vvec[l] & 127)
            ids16 = plsc.load_gather(
                adjN, [(lanes * 0 + 2 * j) + (lanes >> 3), lanes & 7, lvec])
            dst[j >> 3, j & 7] = ids16
          return carry
        lax.fori_loop(0, NG, extract_adj, 0)
    pl.run_scoped(adj_phase, pltpu.VMEM((2 * BPW, 8, 128), jnp.int32))
    def drain_rows(j, carry):
      tj, sj = j >> 3, j & 7
      pltpu.make_async_copy(usr_h.at[0, 0], ueb.at[tj, sj], sem_r).wait()
      pltpu.make_async_copy(ent_h.at[0, 0], vsb.at[tj, sj], sem_r).wait()
      return carry
    lax.fori_loop(0, BPW, drain_rows, 0)

    def ne_phase(neb):
      for sem_n, lo in ((sem_n0, 0), (sem_n1, 16)):
        def fetch_ne(j, carry, sem_n=sem_n):
          row16 = aeb[j >> 3, j & 7]
          for k in range(K):
            e = row16[k]
            r = j * K + k
            pltpu.async_copy(ent_h.at[e >> 3, e & 7],
                             neb.at[r >> 3, r & 7], sem_n)
          return carry
        lax.fori_loop(lo, lo + 16, fetch_ne, 0)
      pl.run_scoped(
          lambda relT_v, wT_v, b_v: _sc_math(
              aeb, arb, ueb, vsb, neb, uet, xt, res,
              relT_v, wT_v, b_v, relT_h, wT_h, b_h, ent_h, out_h, base,
              (sem_n0, sem_n1)),
          pltpu.VMEM((D, NUM_REL), jnp.float32),
          pltpu.VMEM((D, D), jnp.float32),
          pltpu.VMEM((D,), jnp.float32),
      )
    pl.run_scoped(ne_phase,
                  pltpu.VMEM((BPW * K // 8, 8, D), jnp.float32))

  def _sc_math(aeb, arb, ueb, vsb, neb, uet, xt, res,
               relT_v, wT_v, b_v, relT_h, wT_h, b_h, ent_h, out_h, base,
               sem_ns):
    pltpu.sync_copy(relT_h, relT_v)
    pltpu.sync_copy(wT_h, wT_v)
    pltpu.sync_copy(b_h, b_v)
    lanes = lax.iota(jnp.int32, 16)
    for g in range(NG):
      def drain_ne(j, carry):
        for k in range(K):
          r = j * K + k
          pltpu.make_async_copy(ent_h.at[0, 0], neb.at[r >> 3, r & 7],
                                sem_ns[g]).wait()
        return carry
      lax.fori_loop(g * 16, (g + 1) * 16, drain_ne, 0)
      bvec = lanes + g * 16
      bt, bs = bvec >> 3, bvec & 7
      relids = [plsc.load_gather(arb, [bt, bs, lanes * 0 + k])
                for k in range(K)]
      def logits_step(d, s):
        dvec = lanes * 0 + d
        ued = plsc.load_gather(ueb, [bt, bs, dvec])
        uet[d] = ued
        return tuple(
            s[k] + ued * plsc.load_gather(relT_v, [dvec, relids[k]])
            for k in range(K))
      s = lax.fori_loop(
          0, D, logits_step,
          tuple(jnp.zeros((16,), jnp.float32) for _ in range(K)))
      m = s[0]
      for k in range(1, K):
        m = jnp.maximum(m, s[k])
      es = [jnp.exp(s[k] - m) for k in range(K)]
      tot = es[0]
      for k in range(1, K):
        tot = tot + es[k]
      inv = 1.0 / tot
      p = [es[k] * inv for k in range(K)]
      nrows = [bvec * K + k for k in range(K)]
      nts = [(nrows[k] >> 3, nrows[k] & 7) for k in range(K)]
      def wsum_step(d, carry):
        dvec = lanes * 0 + d
        acc = plsc.load_gather(vsb, [bt, bs, dvec])
        for k in range(K):
          ned = plsc.load_gather(neb, [nts[k][0], nts[k][1], dvec])
          acc = acc + p[k] * ned
        xt[d] = acc
        return carry
      lax.fori_loop(0, D, wsum_step, 0)
      def lin_step(do, y):
        dovec = lanes * 0 + do
        accw = plsc.load_gather(b_v, [dovec])
        for j in range(D):
          wv = plsc.load_gather(wT_v, [lanes * 0 + j, dovec])
          accw = accw + xt[j] * wv
        vu = jnp.maximum(accw, 0.0)
        return y + uet[do] * vu
      y = lax.fori_loop(0, D, lin_step, jnp.zeros((16,), jnp.float32))
      res[pl.ds(g * 16, 16)] = 1.0 / (1.0 + jnp.exp(-y))
    pltpu.sync_copy(res, out_h.at[pl.ds(base, BPW)])

  return sc_fused


def kernel(u, v, adj_ent, adj_rel, usr_emb, ent_emb, rel_emb, W, b):
  aeT3 = adj_ent.astype(jnp.int32).T.reshape(2, 8, NUM_ENT)
  arT3 = adj_rel.astype(jnp.int32).T.reshape(2, 8, NUM_ENT)
  usr3 = usr_emb.reshape(NUM_USR // 8, 8, D)
  ent3 = ent_emb.reshape(NUM_ENT // 8, 8, D)
  return _sc_fused_kernel()(
      u.astype(jnp.int32), v.astype(jnp.int32), aeT3, arT3, usr3, ent3,
      rel_emb.T, W.T, b)
